# baseline (speedup 1.0000x reference)
import jax
import jax.numpy as jnp
from jax import lax
from jax.experimental import pallas as pl
from jax.experimental.pallas import tpu as pltpu
from jax.experimental.pallas import tpu_sc as plsc

N_EDGES = 1_600_000
EMB_DIM = 32
PER_W = N_EDGES // 32
CH = 80
NCH = PER_W // CH


def _body(idx_hbm, table_hbm, out_hbm, ridx, rows3, osem):
    c = lax.axis_index("c")
    s = lax.axis_index("s")
    ebase = (s * 2 + c) * PER_W
    iota = lax.iota(jnp.int32, 16)

    def fill(p, carry):
        base = ebase + p * CH
        for k in range(CH // 16):
            ridx[p, pl.ds(k * 16, 16)] = base + k * 16 + iota
        return carry

    lax.fori_loop(0, NCH, fill, 0)

    def scat(q):
        return pltpu.make_async_copy(rows3, out_hbm.at[ridx.at[q]], osem)

    def fire(q, carry):
        scat(q).start()
        return carry

    lax.fori_loop(0, NCH, fire, 0)

    def drain(q, carry):
        scat(q).wait()
        return carry

    lax.fori_loop(0, NCH, drain, 0)


_sc_gather = pl.kernel(
    _body,
    out_type=jax.ShapeDtypeStruct((N_EDGES, 2, 16), jnp.float32),
    mesh=plsc.VectorSubcoreMesh(core_axis_name="c", subcore_axis_name="s"),
    compiler_params=pltpu.CompilerParams(use_tc_tiling_on_sc=False),
    scratch_types=[
        pltpu.VMEM((NCH, CH), jnp.int32),
        pltpu.VMEM((CH, 2, 16), jnp.float32),
        pltpu.SemaphoreType.DMA,
    ],
)


def kernel(edge_attr, edge_index, table):
    del edge_index
    idx = edge_attr.astype(jnp.int32)
    out3 = _sc_gather(idx, table.reshape(-1))
    return out3.reshape(N_EDGES, EMB_DIM)


# speedup vs baseline: 5.1848x; 5.1848x over previous
import jax
import jax.numpy as jnp
from jax import lax
from jax.experimental import pallas as pl
from jax.experimental.pallas import tpu as pltpu
from jax.experimental.pallas import tpu_sc as plsc

N_EDGES = 1_600_000
EMB_DIM = 32
NUM_TYPES = 32
NUM_WORKERS = 32
PER_W = N_EDGES // NUM_WORKERS
GROUP = 1000
NG = PER_W // GROUP
NBUF = 2
UNROLL = 4
GW = GROUP * EMB_DIM


def _body(idx_hbm, table_hbm, out_hbm, table_v, *bufs):
    idxb = bufs[0:NBUF]
    rows = bufs[NBUF:2 * NBUF]
    isem = bufs[2 * NBUF:3 * NBUF]
    osem = bufs[3 * NBUF:4 * NBUF]
    c = lax.axis_index("c")
    s = lax.axis_index("s")
    wid = s * 2 + c
    ebase = wid * PER_W

    def idx_copy(g, b):
        return pltpu.make_async_copy(
            idx_hbm.at[pl.ds(ebase + g * GROUP, GROUP)], idxb[b], isem[b])

    def out_copy(g, b):
        return pltpu.make_async_copy(
            rows[b], out_hbm.at[pl.ds((ebase + g * GROUP) * EMB_DIM, GW)],
            osem[b])

    pltpu.sync_copy(table_hbm, table_v)
    for b in range(NBUF):
        idx_copy(b, b).start()

    def step(g, b):
        idx_copy(g, b).wait()

        @pl.when(g >= NBUF)
        def _():
            out_copy(g - NBUF, b).wait()

        def do16(e0):
            ivec = idxb[b][pl.ds(e0, 16)] * EMB_DIM
            o16 = e0 * EMB_DIM
            for k in range(16):
                base = ivec[k]
                o = o16 + k * EMB_DIM
                rows[b][pl.ds(o, 16)] = table_v[pl.ds(base, 16)]
                rows[b][pl.ds(o + 16, 16)] = table_v[pl.ds(base + 16, 16)]

        @plsc.parallel_loop(0, GROUP // 16, unroll=UNROLL)
        def _(q):
            do16(q * 16)

        if GROUP % 16:
            do16(GROUP - 16)

        out_copy(g, b).start()

        @pl.when(g + NBUF < NG)
        def _():
            idx_copy(g + NBUF, b).start()

    def ring(p, carry):
        for r in range(NBUF):
            step(p * NBUF + r, r)
        return carry

    lax.fori_loop(0, NG // NBUF, ring, 0)

    for b in range(NBUF):
        out_copy(NG - NBUF + b, b).wait()


_sc_gather = pl.kernel(
    _body,
    out_type=jax.ShapeDtypeStruct((N_EDGES * EMB_DIM,), jnp.float32),
    mesh=plsc.VectorSubcoreMesh(core_axis_name="c", subcore_axis_name="s"),
    compiler_params=pltpu.CompilerParams(use_tc_tiling_on_sc=False),
    scratch_types=(
        [pltpu.VMEM((NUM_TYPES * EMB_DIM,), jnp.float32)]
        + [pltpu.VMEM((GROUP,), jnp.int32) for _ in range(NBUF)]
        + [pltpu.VMEM((GW,), jnp.float32) for _ in range(NBUF)]
        + [pltpu.SemaphoreType.DMA for _ in range(2 * NBUF)]
    ),
)


def kernel(edge_attr, edge_index, table):
    del edge_index
    idx = edge_attr.astype(jnp.int32)
    flat = _sc_gather(idx, table.reshape(-1))
    return flat.reshape(N_EDGES, EMB_DIM)
